# baseline (speedup 1.0000x reference)
import jax
import jax.numpy as jnp
from jax.experimental import pallas as pl
from jax.experimental.pallas import tpu as pltpu

N = 10000
K = 32
FEAT = 128
H1, H2 = 64, 32
NUM_CLASS = 40
ALPHA = 0.1
BN = 200

_EPS2 = 1e-24
_NEG = -3.0e38


def _ksum8(a):
    k = a.shape[1]
    while k > 1:
        h = k // 2
        a = a[:, :h, :] + a[:, h:, :]
        k = h
    return a[:, 0, :]


def _body(x_ref, nb_ref, w1d_ref, onesn_ref, b1_ref, w1f_ref, w2d_ref,
          b2_ref, w2f_ref, wc_ref, bc_ref, out_ref):
    f32 = jnp.float32
    bf16 = jnp.bfloat16
    one_m_a = f32(1.0 - ALPHA)

    nb = nb_ref[...]
    nbb = nb.astype(bf16)
    sqb = nbb * nbb

    m = jnp.dot(sqb, onesn_ref[...],
                preferred_element_type=f32)
    t = jnp.dot(nbb, w1d_ref[...],
                preferred_element_type=f32)
    lane1 = jax.lax.broadcasted_iota(jnp.int32, (1, 2 * H1), 1)
    mask1 = jnp.where(lane1 < H1, f32(_NEG), f32(0.0))
    u_dup = t * jax.lax.rsqrt(m) + b1_ref[...]
    d = jnp.maximum(u_dup, mask1)
    sr1 = _ksum8(d.reshape(BN, K, 2 * H1))
    s1 = sr1[:, :H1]
    r1 = sr1[:, H1:]

    ub = d[:, :H1].astype(bf16)
    v = (jnp.dot(ub, w2d_ref[...], preferred_element_type=f32)
         + b2_ref[...])
    rv = jnp.maximum(v, 0.0)
    r2 = _ksum8(rv.reshape(BN, K, H2))
    s2 = (jnp.dot(s1, w2f_ref[...], preferred_element_type=f32)
          + f32(K) * b2_ref[...])

    xb = x_ref[...]
    xinv = jax.lax.rsqrt(jnp.maximum(jnp.sum(xb * xb, axis=1, keepdims=True),
                                     _EPS2))
    h = (jnp.dot(xb, w1f_ref[...], preferred_element_type=f32) * xinv
         + b1_ref[:, :H1])
    x1 = jnp.maximum(h + one_m_a * s1, 0.0)
    x2 = one_m_a * (x1 + r1) + f32(ALPHA) * h
    h2 = (jnp.dot(x2, w2f_ref[...], preferred_element_type=f32)
          + b2_ref[:, :H2])
    x3 = jnp.maximum(h2 + one_m_a * s2, 0.0)
    x4 = one_m_a * (x3 + r2) + f32(ALPHA) * h2
    out_ref[...] = (jnp.dot(x4, wc_ref[...], preferred_element_type=f32)
                    + bc_ref[...])


def kernel(x, neighbor, W1, b1, W2, b2, Wc, bc):
    bf16 = jnp.bfloat16
    nb_flat = neighbor.reshape(N * K, FEAT)
    w1t = W1.T
    w2t = W2.T
    wct = Wc.T
    w1d = jnp.concatenate([w1t, w1t], axis=1).astype(bf16)
    w2d = w2t.astype(bf16)
    onesn = jnp.ones((FEAT, 2 * H1), dtype=bf16)
    b1d = jnp.concatenate([b1, b1]).reshape(1, 2 * H1)
    b2d = b2.reshape(1, H2)
    bcr = bc.reshape(1, NUM_CLASS)

    grid = (N // BN,)
    rep = lambda i: (0, 0)
    out = pl.pallas_call(
        _body,
        grid=grid,
        in_specs=[
            pl.BlockSpec((BN, FEAT), lambda i: (i, 0)),
            pl.BlockSpec((BN * K, FEAT), lambda i: (i, 0)),
            pl.BlockSpec((FEAT, 2 * H1), rep),
            pl.BlockSpec((FEAT, 2 * H1), rep),
            pl.BlockSpec((1, 2 * H1), rep),
            pl.BlockSpec((FEAT, H1), rep),
            pl.BlockSpec((H1, H2), rep),
            pl.BlockSpec((1, H2), rep),
            pl.BlockSpec((H1, H2), rep),
            pl.BlockSpec((H2, NUM_CLASS), rep),
            pl.BlockSpec((1, NUM_CLASS), rep),
        ],
        out_specs=pl.BlockSpec((BN, NUM_CLASS), lambda i: (i, 0)),
        out_shape=jax.ShapeDtypeStruct((N, NUM_CLASS), jnp.float32),
        compiler_params=pltpu.CompilerParams(
            dimension_semantics=("parallel",)),
    )(x, nb_flat, w1d, onesn, b1d, w1t, w2d, b2d, w2t, wct, bcr)
    return out


# speedup vs baseline: 1.0957x; 1.0957x over previous
import jax
import jax.numpy as jnp
from jax.experimental import pallas as pl
from jax.experimental.pallas import tpu as pltpu

N = 10000
K = 32
FEAT = 128
H1, H2 = 64, 32
NUM_CLASS = 40
ALPHA = 0.1
BN = 400

_EPS2 = 1e-24
_NEG = -3.0e38


def _ksum8(a):
    k = a.shape[1]
    while k > 1:
        h = k // 2
        a = a[:, :h, :] + a[:, h:, :]
        k = h
    return a[:, 0, :]


def _body(x_ref, nb_ref, w1d_ref, onesn_ref, b1_ref, w1f_ref, w2d_ref,
          b2_ref, w2f_ref, wc_ref, bc_ref, out_ref):
    f32 = jnp.float32
    bf16 = jnp.bfloat16
    one_m_a = f32(1.0 - ALPHA)

    nb = nb_ref[...]
    nbb = nb.astype(bf16)
    sqb = nbb * nbb

    m = jnp.dot(sqb, onesn_ref[...],
                preferred_element_type=f32)
    t = jnp.dot(nbb, w1d_ref[...],
                preferred_element_type=f32)
    lane1 = jax.lax.broadcasted_iota(jnp.int32, (1, 2 * H1), 1)
    mask1 = jnp.where(lane1 < H1, f32(_NEG), f32(0.0))
    u_dup = t * jax.lax.rsqrt(m) + b1_ref[...]
    d = jnp.maximum(u_dup, mask1)
    sr1 = _ksum8(d.reshape(BN, K, 2 * H1))
    s1 = sr1[:, :H1]
    r1 = sr1[:, H1:]

    ub = d[:, :H1].astype(bf16)
    v = (jnp.dot(ub, w2d_ref[...], preferred_element_type=f32)
         + b2_ref[...])
    rv = jnp.maximum(v, 0.0)
    r2 = _ksum8(rv.reshape(BN, K, H2))
    s2 = (jnp.dot(s1, w2f_ref[...], preferred_element_type=f32)
          + f32(K) * b2_ref[...])

    xb = x_ref[...]
    xinv = jax.lax.rsqrt(jnp.maximum(jnp.sum(xb * xb, axis=1, keepdims=True),
                                     _EPS2))
    h = (jnp.dot(xb, w1f_ref[...], preferred_element_type=f32) * xinv
         + b1_ref[:, :H1])
    x1 = jnp.maximum(h + one_m_a * s1, 0.0)
    x2 = one_m_a * (x1 + r1) + f32(ALPHA) * h
    h2 = (jnp.dot(x2, w2f_ref[...], preferred_element_type=f32)
          + b2_ref[:, :H2])
    x3 = jnp.maximum(h2 + one_m_a * s2, 0.0)
    x4 = one_m_a * (x3 + r2) + f32(ALPHA) * h2
    out_ref[...] = (jnp.dot(x4, wc_ref[...], preferred_element_type=f32)
                    + bc_ref[...])


def kernel(x, neighbor, W1, b1, W2, b2, Wc, bc):
    bf16 = jnp.bfloat16
    nb_flat = neighbor.reshape(N * K, FEAT)
    w1t = W1.T
    w2t = W2.T
    wct = Wc.T
    w1d = jnp.concatenate([w1t, w1t], axis=1).astype(bf16)
    w2d = w2t.astype(bf16)
    onesn = jnp.ones((FEAT, 2 * H1), dtype=bf16)
    b1d = jnp.concatenate([b1, b1]).reshape(1, 2 * H1)
    b2d = b2.reshape(1, H2)
    bcr = bc.reshape(1, NUM_CLASS)

    grid = (N // BN,)
    rep = lambda i: (0, 0)
    out = pl.pallas_call(
        _body,
        grid=grid,
        in_specs=[
            pl.BlockSpec((BN, FEAT), lambda i: (i, 0)),
            pl.BlockSpec((BN * K, FEAT), lambda i: (i, 0)),
            pl.BlockSpec((FEAT, 2 * H1), rep),
            pl.BlockSpec((FEAT, 2 * H1), rep),
            pl.BlockSpec((1, 2 * H1), rep),
            pl.BlockSpec((FEAT, H1), rep),
            pl.BlockSpec((H1, H2), rep),
            pl.BlockSpec((1, H2), rep),
            pl.BlockSpec((H1, H2), rep),
            pl.BlockSpec((H2, NUM_CLASS), rep),
            pl.BlockSpec((1, NUM_CLASS), rep),
        ],
        out_specs=pl.BlockSpec((BN, NUM_CLASS), lambda i: (i, 0)),
        out_shape=jax.ShapeDtypeStruct((N, NUM_CLASS), jnp.float32),
        compiler_params=pltpu.CompilerParams(
            dimension_semantics=("parallel",)),
    )(x, nb_flat, w1d, onesn, b1d, w1t, w2d, b2d, w2t, wct, bcr)
    return out
